# Initial kernel scaffold; baseline (speedup 1.0000x reference)
import jax
import jax.numpy as jnp
from jax.experimental import pallas as pl


def kernel(numeric_inputs, categorical_inputs, W_linear, b_linear, numeric_table, cat_tables, W1, b1, W2, b2, Wout, bout):
    raise NotImplementedError("write your pallas kernel here")


# speedup vs baseline: 1.4820x; 1.4820x over previous
import functools

import jax
import jax.numpy as jnp
from jax import lax
from jax.experimental import pallas as pl
from jax.experimental.pallas import tpu as pltpu
from jax.experimental.pallas import tpu_sc as plsc

B = 1024
NUM_NUM = 13
N_CAT = 26
CAT_VOCAB = 1000
D = 64
NC = 2
NS = 16
NW = NC * NS
RW = B // NW
CPW = RW * N_CAT
NPW = RW * NUM_NUM
CCH = 7
NCH = 4


def _sc_body(catidx_hbm, numidx_hbm, ctab_hbm, ntab_hbm, wcat_hbm,
             fm_hbm, lin_hbm,
             cidx, nidx, crows, nrows, wvals, fmv, linv, sem):
    wid = lax.axis_index("s") * NC + lax.axis_index("c")
    base = wid * RW

    pltpu.sync_copy(catidx_hbm.at[wid], cidx)
    pltpu.sync_copy(numidx_hbm.at[wid], nidx)

    copies = []
    for c in range(CCH):
        copies.append(pltpu.async_copy(
            ctab_hbm.at[cidx.at[c]], crows.at[pl.ds(c * 128, 128)], sem))
    for c in range(NCH):
        copies.append(pltpu.async_copy(
            ntab_hbm.at[nidx.at[c]], nrows.at[pl.ds(c * 128, 128)], sem))
    for c in range(CCH):
        copies.append(pltpu.async_copy(
            wcat_hbm.at[cidx.at[c]], wvals.at[pl.ds(c * 128, 128)], sem))
    for cp in copies:
        cp.wait()

    la = jnp.zeros((16,), jnp.float32)
    lb = jnp.zeros((16,), jnp.float32)
    for f in range(N_CAT):
        la = la + wvals[pl.ds(f * RW, 16)]
        lb = lb + wvals[pl.ds(f * RW + 16, 16)]
    linv[pl.ds(0, 16)] = la
    linv[pl.ds(16, 16)] = lb
    pltpu.sync_copy(linv, lin_hbm.at[pl.ds(base, RW)])

    zero = jnp.zeros((16,), jnp.float32)

    def row_body(r, _):
        def acc(rows_ref, row, carry):
            s0, s1, s2, s3, q0, q1, q2, q3 = carry
            v0 = rows_ref[row, pl.ds(0, 16)]
            v1 = rows_ref[row, pl.ds(16, 16)]
            v2 = rows_ref[row, pl.ds(32, 16)]
            v3 = rows_ref[row, pl.ds(48, 16)]
            return (s0 + v0, s1 + v1, s2 + v2, s3 + v3,
                    q0 + v0 * v0, q1 + v1 * v1, q2 + v2 * v2, q3 + v3 * v3)

        carry = lax.fori_loop(
            0, N_CAT, lambda f, cr: acc(crows, f * RW + r, cr), (zero,) * 8)
        carry = lax.fori_loop(
            0, NUM_NUM, lambda k, cr: acc(nrows, k * RW + r, cr), carry)
        s0, s1, s2, s3, q0, q1, q2, q3 = carry
        fmv[r, pl.ds(0, 16)] = 0.5 * (s0 * s0 - q0)
        fmv[r, pl.ds(16, 16)] = 0.5 * (s1 * s1 - q1)
        fmv[r, pl.ds(32, 16)] = 0.5 * (s2 * s2 - q2)
        fmv[r, pl.ds(48, 16)] = 0.5 * (s3 * s3 - q3)
        return 0

    lax.fori_loop(0, RW, row_body, 0)
    pltpu.sync_copy(fmv, fm_hbm.at[pl.ds(base, RW)])


_sc_call = pl.kernel(
    _sc_body,
    out_type=(
        jax.ShapeDtypeStruct((B, D), jnp.float32),
        jax.ShapeDtypeStruct((B,), jnp.float32),
    ),
    mesh=plsc.VectorSubcoreMesh(core_axis_name="c", subcore_axis_name="s"),
    scratch_types=[
        pltpu.VMEM((CCH, 128), jnp.int32),
        pltpu.VMEM((NCH, 128), jnp.int32),
        pltpu.VMEM((CCH * 128, D), jnp.float32),
        pltpu.VMEM((NCH * 128, D), jnp.float32),
        pltpu.VMEM((CCH * 128,), jnp.float32),
        pltpu.VMEM((RW, D), jnp.float32),
        pltpu.VMEM((RW,), jnp.float32),
        pltpu.SemaphoreType.DMA,
    ],
    compiler_params=pltpu.CompilerParams(use_tc_tiling_on_sc=False),
)


def _tc_body(fm_ref, lin_ref, numf_ref, wnum_ref, w1_ref, b1_ref, w2_ref,
             b2_ref, woutt_ref, bsum_ref, out_ref):
    x = jnp.dot(fm_ref[...], w1_ref[...], preferred_element_type=jnp.float32)
    x = jnp.maximum(x + b1_ref[...], 0.0)
    x = jnp.dot(x, w2_ref[...], preferred_element_type=jnp.float32)
    x = jnp.maximum(x + b2_ref[...], 0.0)
    inter = jnp.sum(x * woutt_ref[...], axis=1, keepdims=True)
    numlin = jnp.sum(numf_ref[...] * wnum_ref[...], axis=1, keepdims=True)
    out_ref[...] = inter + lin_ref[...] + numlin + bsum_ref[0, 0]


def kernel(numeric_inputs, categorical_inputs, W_linear, b_linear,
           numeric_table, cat_tables, W1, b1, W2, b2, Wout, bout):
    cat_gidx = categorical_inputs + (
        jnp.arange(N_CAT, dtype=jnp.int32) * CAT_VOCAB)[None, :]
    cat_gidx = cat_gidx.reshape(NW, RW, N_CAT).transpose(0, 2, 1)
    cat_gidx = cat_gidx.reshape(NW, CPW)
    cat_gidx = jnp.pad(cat_gidx, ((0, 0), (0, CCH * 128 - CPW)))
    cat_gidx = cat_gidx.reshape(NW, CCH, 128)

    num_gidx = numeric_inputs.reshape(NW, RW, NUM_NUM).transpose(0, 2, 1)
    num_gidx = num_gidx.reshape(NW, NPW)
    num_gidx = jnp.pad(num_gidx, ((0, 0), (0, NCH * 128 - NPW)))
    num_gidx = num_gidx.reshape(NW, NCH, 128)

    cat_flat = cat_tables.reshape(N_CAT * CAT_VOCAB, D)
    wcat = W_linear[NUM_NUM:, 0]

    fm, lin = _sc_call(cat_gidx, num_gidx, cat_flat, numeric_table, wcat)

    out = pl.pallas_call(
        _tc_body,
        out_shape=jax.ShapeDtypeStruct((B, 1), jnp.float32),
    )(
        fm,
        lin.reshape(B, 1),
        numeric_inputs.astype(jnp.float32),
        W_linear[:NUM_NUM, 0].reshape(1, NUM_NUM),
        W1,
        b1.reshape(1, -1),
        W2,
        b2.reshape(1, -1),
        Wout.reshape(1, -1),
        (b_linear + bout).reshape(1, 1),
    )
    return out
